# Initial kernel scaffold; baseline (speedup 1.0000x reference)
import jax
import jax.numpy as jnp
from jax.experimental import pallas as pl


def kernel(x_user, x_item, linW, linB, kW, kB, qW, qB, vW, vB, aW, aB, skip, relA, relM, relP, bnG, bnB, ei_ui, ei_iu):
    raise NotImplementedError("write your pallas kernel here")


# speedup vs baseline: 12.1452x; 12.1452x over previous
import jax
import jax.numpy as jnp
from jax import lax
from jax.experimental import pallas as pl
from jax.experimental.pallas import tpu as pltpu
from jax.experimental.pallas import tpu_sc as plsc

N = 10000
E = 160000
D_IN = 128
H = 8
HID = 256
DH = 32
NLAYER = 2
MW = 144
C = 40
NTILE = 16
EPT = E // NTILE
NCH = EPT // C
NP = 10240
RPT = NP // NTILE
R = 2000


def _k1_body(x_ref, w_ref, b_ref, o_ref):
    o_ref[0] = jax.nn.relu(
        jnp.dot(x_ref[0], w_ref[0], preferred_element_type=jnp.float32)
        + b_ref[0])


def _k1(x, w, b):
    return pl.pallas_call(
        _k1_body,
        grid=(2, N // R),
        in_specs=[
            pl.BlockSpec((1, R, D_IN), lambda t, r: (t, r, 0)),
            pl.BlockSpec((1, D_IN, HID), lambda t, r: (t, 0, 0)),
            pl.BlockSpec((1, 1, HID), lambda t, r: (t, 0, 0)),
        ],
        out_specs=pl.BlockSpec((1, R, HID), lambda t, r: (t, r, 0)),
        out_shape=jax.ShapeDtypeStruct((2, N, HID), jnp.float32),
    )(x, w, b)


def _k2_body(x_ref, wq_ref, wk_ref, wv_ref, bq_ref, bk_ref, bv_ref,
             q_ref, k_ref, v_ref):
    x = x_ref[0]
    q_ref[0] = jnp.dot(x, wq_ref[0], preferred_element_type=jnp.float32) + bq_ref[0]
    k_ref[0] = jnp.dot(x, wk_ref[0], preferred_element_type=jnp.float32) + bk_ref[0]
    v_ref[0] = jnp.dot(x, wv_ref[0], preferred_element_type=jnp.float32) + bv_ref[0]


def _k2(x, wq, wk, wv, bq, bk, bv):
    wspec = pl.BlockSpec((1, HID, HID), lambda t, r: (t, 0, 0))
    bspec = pl.BlockSpec((1, 1, HID), lambda t, r: (t, 0, 0))
    ospec = pl.BlockSpec((1, R, HID), lambda t, r: (t, r, 0))
    oshape = jax.ShapeDtypeStruct((2, N, HID), jnp.float32)
    return pl.pallas_call(
        _k2_body,
        grid=(2, N // R),
        in_specs=[pl.BlockSpec((1, R, HID), lambda t, r: (t, r, 0)),
                  wspec, wspec, wspec, bspec, bspec, bspec],
        out_specs=[ospec, ospec, ospec],
        out_shape=[oshape, oshape, oshape],
    )(x, wq, wk, wv, bq, bk, bv)


def _k3_body(acc_ref, x_ref, w_ref, b_ref, al_ref, g_ref, bb_ref, o_ref):
    a0 = acc_ref[0, 0]
    a1 = acc_ref[0, 1]
    num = jnp.concatenate([a0[:, 0:128], a1[:, 0:128]], axis=1)
    den8 = jnp.concatenate([a0[:, 128:132], a1[:, 128:132]], axis=1)
    r8 = lax.broadcasted_iota(jnp.int32, (8, HID), 0)
    c8 = lax.broadcasted_iota(jnp.int32, (8, HID), 1) // DH
    expand = (r8 == c8).astype(jnp.float32)
    den = jnp.dot(den8, expand, preferred_element_type=jnp.float32)
    o = num / (den + 1e-16)
    o = 0.5 * o * (1.0 + lax.erf(o * 0.7071067811865475))
    o = jnp.dot(o, w_ref[0], preferred_element_type=jnp.float32) + b_ref[0]
    al = al_ref[0]
    o = al * o + (1.0 - al) * x_ref[0]
    o_ref[0] = g_ref[0] * o + bb_ref[0]


def _k3(acc, x, w, b, al, bng, bnb):
    return pl.pallas_call(
        _k3_body,
        grid=(2, N // R),
        in_specs=[
            pl.BlockSpec((1, 2, R, MW), lambda t, r: (t, 0, r, 0)),
            pl.BlockSpec((1, R, HID), lambda t, r: (t, r, 0)),
            pl.BlockSpec((1, HID, HID), lambda t, r: (t, 0, 0)),
            pl.BlockSpec((1, 1, HID), lambda t, r: (t, 0, 0)),
            pl.BlockSpec((1, 1, 1), lambda t, r: (t, 0, 0)),
            pl.BlockSpec((1, 1, HID), lambda t, r: (0, 0, 0)),
            pl.BlockSpec((1, 1, HID), lambda t, r: (0, 0, 0)),
        ],
        out_specs=pl.BlockSpec((1, R, HID), lambda t, r: (t, r, 0)),
        out_shape=jax.ShapeDtypeStruct((2, N, HID), jnp.float32),
    )(acc, x, w, b, al, bng, bnb)


def _edge_body(q_hbm, kt_hbm, vt_hbm, si_hbm, di_hbm, out_hbm,
               acc_sh, si_v, di_v, ixd_v, ixs_v, dsc_v, qr, ktr, vtr, msg,
               sem0, sem1, sem2):
    c = lax.axis_index("c")
    s = lax.axis_index("s")

    lane = lax.iota(jnp.int32, 16)
    oh = [jnp.where(lane == h, 1.0, 0.0).astype(jnp.float32) for h in range(4)]

    z16 = jnp.zeros((16,), jnp.float32)

    def zrow(r, carry):
        for k2 in range(MW // 16):
            msg[r, pl.ds(k2 * 16, 16)] = z16
        return carry

    lax.fori_loop(0, C, zrow, 0)
    for j in range(RPT // C):
        pltpu.sync_copy(msg, acc_sh.at[pl.ds(s * RPT + j * C, C)])
    plsc.subcore_barrier()

    base = s * EPT

    def chunk(ch, carry):
        off = base + ch * C
        pltpu.sync_copy(si_hbm.at[pl.ds(off, 48)], si_v)
        pltpu.sync_copy(di_hbm.at[pl.ds(off, 48)], di_v)
        for lo in (0, 16, 24):
            sv = si_v[pl.ds(lo, 16)]
            dv = di_v[pl.ds(lo, 16)]
            ixs_v[pl.ds(lo, 16)] = sv + sv + c
            ixd_v[pl.ds(lo, 16)] = dv + dv + c
            dsc_v[pl.ds(lo, 16)] = dv
        cp_q = pltpu.async_copy(q_hbm.at[ixd_v], qr, sem0)
        cp_k = pltpu.async_copy(kt_hbm.at[ixs_v], ktr, sem1)
        cp_v = pltpu.async_copy(vt_hbm.at[ixs_v], vtr, sem2)
        cp_q.wait()
        cp_k.wait()
        cp_v.wait()

        def edge(e, ecarry):
            dsum = None
            for h in range(4):
                qa = qr[e, pl.ds(h * 32, 16)]
                qb = qr[e, pl.ds(h * 32 + 16, 16)]
                ka = ktr[e, pl.ds(h * 32, 16)]
                kb = ktr[e, pl.ds(h * 32 + 16, 16)]
                sc = jnp.sum(qa * ka + qb * kb)
                w = jnp.exp(jnp.full((16,), sc, jnp.float32))
                msg[e, pl.ds(h * 32, 16)] = w * vtr[e, pl.ds(h * 32, 16)]
                msg[e, pl.ds(h * 32 + 16, 16)] = w * vtr[e, pl.ds(h * 32 + 16, 16)]
                part = w * oh[h]
                dsum = part if dsum is None else dsum + part
            msg[e, pl.ds(128, 16)] = dsum
            return ecarry

        lax.fori_loop(0, C, edge, 0)
        pltpu.sync_copy(msg, acc_sh.at[dsc_v], add=True)
        return carry

    lax.fori_loop(0, NCH, chunk, 0)
    plsc.subcore_barrier()
    pltpu.sync_copy(acc_sh.at[pl.ds(s * RPT, RPT)],
                    out_hbm.at[c, pl.ds(s * RPT, RPT)])


_edge_call = pl.kernel(
    _edge_body,
    out_type=jax.ShapeDtypeStruct((2, NP, MW), jnp.float32),
    mesh=plsc.VectorSubcoreMesh(core_axis_name="c", subcore_axis_name="s"),
    compiler_params=pltpu.CompilerParams(use_tc_tiling_on_sc=False,
                                         needs_layout_passes=False),
    scratch_types=[
        pltpu.VMEM_SHARED((NP, MW), jnp.float32),
        pltpu.VMEM((48,), jnp.int32),
        pltpu.VMEM((48,), jnp.int32),
        pltpu.VMEM((C,), jnp.int32),
        pltpu.VMEM((C,), jnp.int32),
        pltpu.VMEM((C,), jnp.int32),
        pltpu.VMEM((C, 128), jnp.float32),
        pltpu.VMEM((C, 128), jnp.float32),
        pltpu.VMEM((C, 128), jnp.float32),
        pltpu.VMEM((C, MW), jnp.float32),
        pltpu.SemaphoreType.DMA,
        pltpu.SemaphoreType.DMA,
        pltpu.SemaphoreType.DMA,
    ],
)


def kernel(x_user, x_item, linW, linB, kW, kB, qW, qB, vW, vB, aW, aB,
           skip, relA, relM, relP, bnG, bnB, ei_ui, ei_iu):
    x0 = jnp.stack([x_user, x_item])
    xs = _k1(x0, linW, linB.reshape(2, 1, HID))

    pad = jnp.zeros((64,), jnp.int32)
    si = [jnp.concatenate([ei_ui[0], pad]), jnp.concatenate([ei_iu[0], pad])]
    di = [jnp.concatenate([ei_ui[1], pad]), jnp.concatenate([ei_iu[1], pad])]

    for l in range(NLAYER):
        scale = relP[l] / jnp.sqrt(jnp.float32(DH))
        kWf = jnp.einsum('tihd,thde,th->tihe',
                         kW[l].reshape(2, HID, H, DH), relA[l], scale
                         ).reshape(2, HID, HID)
        kBf = jnp.einsum('thd,thde,th->the',
                         kB[l].reshape(2, H, DH), relA[l], scale
                         ).reshape(2, 1, HID)
        vWf = jnp.einsum('tihd,thde->tihe',
                         vW[l].reshape(2, HID, H, DH), relM[l]
                         ).reshape(2, HID, HID)
        vBf = jnp.einsum('thd,thde->the',
                         vB[l].reshape(2, H, DH), relM[l]
                         ).reshape(2, 1, HID)
        Q, Kt, Vt = _k2(xs, qW[l], kWf, vWf,
                        qB[l].reshape(2, 1, HID), kBf, vBf)
        q2 = Q.reshape(2, 2 * N, 128)
        k2 = Kt.reshape(2, 2 * N, 128)
        v2 = Vt.reshape(2, 2 * N, 128)
        acc_item = _edge_call(q2[1], k2[0], v2[0], si[0], di[0])
        acc_user = _edge_call(q2[0], k2[1], v2[1], si[1], di[1])
        acc = jnp.stack([acc_user, acc_item])
        alpha = jax.nn.sigmoid(skip[l]).reshape(2, 1, 1)
        bng = (bnG[l] / jnp.sqrt(1.0 + 1e-5)).reshape(1, 1, HID)
        bnb = bnB[l].reshape(1, 1, HID)
        xs = _k3(acc, xs, aW[l], aB[l].reshape(2, 1, HID), alpha, bng, bnb)
    return xs


# speedup vs baseline: 16.7944x; 1.3828x over previous
import jax
import jax.numpy as jnp
from jax import lax
from jax.experimental import pallas as pl
from jax.experimental.pallas import tpu as pltpu
from jax.experimental.pallas import tpu_sc as plsc

N = 10000
E = 160000
D_IN = 128
H = 8
HID = 256
DH = 32
NLAYER = 2
MW = 144
C = 40
NTILE = 16
EPT = E // NTILE
NCH = EPT // C
NP = 10240
RPT = NP // NTILE
R = 2000


def _k1_body(x_ref, w_ref, b_ref, o_ref):
    o_ref[0] = jax.nn.relu(
        jnp.dot(x_ref[0], w_ref[0], preferred_element_type=jnp.float32)
        + b_ref[0])


def _k1(x, w, b):
    return pl.pallas_call(
        _k1_body,
        grid=(2, N // R),
        in_specs=[
            pl.BlockSpec((1, R, D_IN), lambda t, r: (t, r, 0)),
            pl.BlockSpec((1, D_IN, HID), lambda t, r: (t, 0, 0)),
            pl.BlockSpec((1, 1, HID), lambda t, r: (t, 0, 0)),
        ],
        out_specs=pl.BlockSpec((1, R, HID), lambda t, r: (t, r, 0)),
        out_shape=jax.ShapeDtypeStruct((2, N, HID), jnp.float32),
    )(x, w, b)


def _k2_body(x_ref, wq_ref, wk_ref, wv_ref, bq_ref, bk_ref, bv_ref,
             q_ref, kv_ref):
    x = x_ref[0]
    q_ref[0] = jnp.dot(x, wq_ref[0], preferred_element_type=jnp.float32) + bq_ref[0]
    k = jnp.dot(x, wk_ref[0], preferred_element_type=jnp.float32) + bk_ref[0]
    v = jnp.dot(x, wv_ref[0], preferred_element_type=jnp.float32) + bv_ref[0]
    kv = jnp.concatenate([k.reshape(R, 2, 128), v.reshape(R, 2, 128)], axis=2)
    kv_ref[0] = kv.reshape(2 * R, HID)


def _k2(x, wq, wk, wv, bq, bk, bv):
    wspec = pl.BlockSpec((1, HID, HID), lambda t, r: (t, 0, 0))
    bspec = pl.BlockSpec((1, 1, HID), lambda t, r: (t, 0, 0))
    return pl.pallas_call(
        _k2_body,
        grid=(2, N // R),
        in_specs=[pl.BlockSpec((1, R, HID), lambda t, r: (t, r, 0)),
                  wspec, wspec, wspec, bspec, bspec, bspec],
        out_specs=[pl.BlockSpec((1, R, HID), lambda t, r: (t, r, 0)),
                   pl.BlockSpec((1, 2 * R, HID), lambda t, r: (t, r, 0))],
        out_shape=[jax.ShapeDtypeStruct((2, N, HID), jnp.float32),
                   jax.ShapeDtypeStruct((2, 2 * N, HID), jnp.float32)],
    )(x, wq, wk, wv, bq, bk, bv)


def _k3_body(acc_ref, x_ref, w_ref, b_ref, al_ref, g_ref, bb_ref, o_ref):
    a0 = acc_ref[0, 0]
    a1 = acc_ref[0, 1]
    num = jnp.concatenate([a0[:, 0:128], a1[:, 0:128]], axis=1)
    den8 = jnp.concatenate([a0[:, 128:132], a1[:, 128:132]], axis=1)
    r8 = lax.broadcasted_iota(jnp.int32, (8, HID), 0)
    c8 = lax.broadcasted_iota(jnp.int32, (8, HID), 1) // DH
    expand = (r8 == c8).astype(jnp.float32)
    den = jnp.dot(den8, expand, preferred_element_type=jnp.float32)
    o = num / (den + 1e-16)
    o = 0.5 * o * (1.0 + lax.erf(o * 0.7071067811865475))
    o = jnp.dot(o, w_ref[0], preferred_element_type=jnp.float32) + b_ref[0]
    al = al_ref[0]
    o = al * o + (1.0 - al) * x_ref[0]
    o_ref[0] = g_ref[0] * o + bb_ref[0]


def _k3(acc, x, w, b, al, bng, bnb):
    return pl.pallas_call(
        _k3_body,
        grid=(2, N // R),
        in_specs=[
            pl.BlockSpec((1, 2, R, MW), lambda t, r: (t, 0, r, 0)),
            pl.BlockSpec((1, R, HID), lambda t, r: (t, r, 0)),
            pl.BlockSpec((1, HID, HID), lambda t, r: (t, 0, 0)),
            pl.BlockSpec((1, 1, HID), lambda t, r: (t, 0, 0)),
            pl.BlockSpec((1, 1, 1), lambda t, r: (t, 0, 0)),
            pl.BlockSpec((1, 1, HID), lambda t, r: (0, 0, 0)),
            pl.BlockSpec((1, 1, HID), lambda t, r: (0, 0, 0)),
        ],
        out_specs=pl.BlockSpec((1, R, HID), lambda t, r: (t, r, 0)),
        out_shape=jax.ShapeDtypeStruct((2, N, HID), jnp.float32),
    )(acc, x, w, b, al, bng, bnb)


def _edge_body(q_hbm, kv_hbm, ei_hbm, out_hbm,
               acc_sh, eib, ixd, ixkv, dsc, qr, kvr, msg,
               semI, semQ, semKV):
    c = lax.axis_index("c")
    s = lax.axis_index("s")

    lane = lax.iota(jnp.int32, 16)
    oh = [jnp.where(lane == h, 1.0, 0.0).astype(jnp.float32) for h in range(4)]

    z16 = jnp.zeros((16,), jnp.float32)

    def zrow(r, carry):
        for k2 in range(MW // 16):
            msg[r, pl.ds(k2 * 16, 16)] = z16
        return carry

    lax.fori_loop(0, C, zrow, 0)
    for j in range(RPT // C):
        pltpu.sync_copy(msg, acc_sh.at[pl.ds(s * RPT + j * C, C)])
    plsc.subcore_barrier()

    base = s * EPT

    def load_idx(ch, b):
        return pltpu.async_copy(
            ei_hbm.at[pl.ds(0, 2), pl.ds(base + ch * C, 48)], eib.at[b], semI)

    def comp_idx(b):
        for lo in (0, 16, 24):
            sv = eib[b, 0, pl.ds(lo, 16)]
            dv = eib[b, 1, pl.ds(lo, 16)]
            ixkv[b, pl.ds(lo, 16)] = sv + sv + c
            ixd[b, pl.ds(lo, 16)] = dv + dv + c
            dsc[b, pl.ds(lo, 16)] = dv

    def start_gathers(b):
        pltpu.async_copy(q_hbm.at[ixd.at[b]], qr.at[b], semQ)
        pltpu.async_copy(kv_hbm.at[ixkv.at[b]], kvr.at[b], semKV)

    load_idx(0, 0).wait()
    comp_idx(0)
    start_gathers(0)
    load_idx(1, 1)

    def chunk(ch, carry):
        b = lax.rem(ch, 2)
        bn = lax.rem(ch + 1, 2)

        @pl.when(ch + 1 < NCH)
        def _():
            pltpu.make_async_copy(
                ei_hbm.at[pl.ds(0, 2), pl.ds(base, 48)], eib.at[bn],
                semI).wait()
            comp_idx(bn)
            start_gathers(bn)

        @pl.when(ch + 2 < NCH)
        def _():
            load_idx(ch + 2, b)

        pltpu.make_async_copy(q_hbm.at[ixd.at[b]], qr.at[b], semQ).wait()
        pltpu.make_async_copy(kv_hbm.at[ixkv.at[b]], kvr.at[b], semKV).wait()

        def edge(e, ecarry):
            dsum = None
            for h in range(4):
                qa = qr[b, e, pl.ds(h * 32, 16)]
                qb = qr[b, e, pl.ds(h * 32 + 16, 16)]
                ka = kvr[b, e, pl.ds(h * 32, 16)]
                kb = kvr[b, e, pl.ds(h * 32 + 16, 16)]
                sc = jnp.sum(qa * ka + qb * kb)
                w = jnp.exp(jnp.full((16,), sc, jnp.float32))
                msg[e, pl.ds(h * 32, 16)] = w * kvr[b, e, pl.ds(128 + h * 32, 16)]
                msg[e, pl.ds(h * 32 + 16, 16)] = w * kvr[b, e, pl.ds(128 + h * 32 + 16, 16)]
                part = w * oh[h]
                dsum = part if dsum is None else dsum + part
            msg[e, pl.ds(128, 16)] = dsum
            return ecarry

        lax.fori_loop(0, C, edge, 0)
        pltpu.sync_copy(msg, acc_sh.at[dsc.at[b]], add=True)
        return carry

    lax.fori_loop(0, NCH, chunk, 0)
    plsc.subcore_barrier()
    pltpu.sync_copy(acc_sh.at[pl.ds(s * RPT, RPT)],
                    out_hbm.at[c, pl.ds(s * RPT, RPT)])


_edge_call = pl.kernel(
    _edge_body,
    out_type=jax.ShapeDtypeStruct((2, NP, MW), jnp.float32),
    mesh=plsc.VectorSubcoreMesh(core_axis_name="c", subcore_axis_name="s"),
    compiler_params=pltpu.CompilerParams(use_tc_tiling_on_sc=False,
                                         needs_layout_passes=False),
    scratch_types=[
        pltpu.VMEM_SHARED((NP, MW), jnp.float32),
        pltpu.VMEM((2, 2, 48), jnp.int32),
        pltpu.VMEM((2, C), jnp.int32),
        pltpu.VMEM((2, C), jnp.int32),
        pltpu.VMEM((2, C), jnp.int32),
        pltpu.VMEM((2, C, 128), jnp.float32),
        pltpu.VMEM((2, C, HID), jnp.float32),
        pltpu.VMEM((C, MW), jnp.float32),
        pltpu.SemaphoreType.DMA,
        pltpu.SemaphoreType.DMA,
        pltpu.SemaphoreType.DMA,
    ],
)


def kernel(x_user, x_item, linW, linB, kW, kB, qW, qB, vW, vB, aW, aB,
           skip, relA, relM, relP, bnG, bnB, ei_ui, ei_iu):
    x0 = jnp.stack([x_user, x_item])
    xs = _k1(x0, linW, linB.reshape(2, 1, HID))

    pad = jnp.zeros((2, 64), jnp.int32)
    eis = [jnp.concatenate([ei_ui, pad], axis=1),
           jnp.concatenate([ei_iu, pad], axis=1)]

    for l in range(NLAYER):
        scale = relP[l] / jnp.sqrt(jnp.float32(DH))
        kWf = jnp.einsum('tihd,thde,th->tihe',
                         kW[l].reshape(2, HID, H, DH), relA[l], scale
                         ).reshape(2, HID, HID)
        kBf = jnp.einsum('thd,thde,th->the',
                         kB[l].reshape(2, H, DH), relA[l], scale
                         ).reshape(2, 1, HID)
        vWf = jnp.einsum('tihd,thde->tihe',
                         vW[l].reshape(2, HID, H, DH), relM[l]
                         ).reshape(2, HID, HID)
        vBf = jnp.einsum('thd,thde->the',
                         vB[l].reshape(2, H, DH), relM[l]
                         ).reshape(2, 1, HID)
        Q, KV = _k2(xs, qW[l], kWf, vWf,
                    qB[l].reshape(2, 1, HID), kBf, vBf)
        q2 = Q.reshape(2, 2 * N, 128)
        acc_item = _edge_call(q2[1], KV[0], eis[0])
        acc_user = _edge_call(q2[0], KV[1], eis[1])
        acc = jnp.stack([acc_user, acc_item])
        alpha = jax.nn.sigmoid(skip[l]).reshape(2, 1, 1)
        bng = (bnG[l] / jnp.sqrt(1.0 + 1e-5)).reshape(1, 1, HID)
        bnb = bnB[l].reshape(1, 1, HID)
        xs = _k3(acc, xs, aW[l], aB[l].reshape(2, 1, HID), alpha, bng, bnb)
    return xs


# speedup vs baseline: 52.7521x; 3.1410x over previous
import jax
import jax.numpy as jnp
from jax import lax
from jax.experimental import pallas as pl
from jax.experimental.pallas import tpu as pltpu
from jax.experimental.pallas import tpu_sc as plsc

N = 10000
E = 160000
D_IN = 128
H = 8
HID = 256
DH = 32
NLAYER = 2
MW = 144
C = 40
NTILE = 16
EPT = E // NTILE
NCH = EPT // C
NP = 10240
RPT = NP // NTILE
R = 2000


def _k1_body(x_ref, w_ref, b_ref, o_ref):
    o_ref[0] = jax.nn.relu(
        jnp.dot(x_ref[0], w_ref[0], preferred_element_type=jnp.float32)
        + b_ref[0])


def _k1(x, w, b):
    return pl.pallas_call(
        _k1_body,
        grid=(2, N // R),
        in_specs=[
            pl.BlockSpec((1, R, D_IN), lambda t, r: (t, r, 0)),
            pl.BlockSpec((1, D_IN, HID), lambda t, r: (t, 0, 0)),
            pl.BlockSpec((1, 1, HID), lambda t, r: (t, 0, 0)),
        ],
        out_specs=pl.BlockSpec((1, R, HID), lambda t, r: (t, r, 0)),
        out_shape=jax.ShapeDtypeStruct((2, N, HID), jnp.float32),
    )(x, w, b)


def _k2_body(x_ref, wq_ref, wk_ref, wv_ref, bq_ref, bk_ref, bv_ref,
             q_ref, kv_ref):
    x = x_ref[0]
    q_ref[0] = jnp.dot(x, wq_ref[0], preferred_element_type=jnp.float32) + bq_ref[0]
    k = jnp.dot(x, wk_ref[0], preferred_element_type=jnp.float32) + bk_ref[0]
    v = jnp.dot(x, wv_ref[0], preferred_element_type=jnp.float32) + bv_ref[0]
    kv = jnp.concatenate([k.reshape(R, 2, 128), v.reshape(R, 2, 128)], axis=2)
    kv_ref[0] = kv.reshape(2 * R, HID)


def _k2(x, wq, wk, wv, bq, bk, bv):
    wspec = pl.BlockSpec((1, HID, HID), lambda t, r: (t, 0, 0))
    bspec = pl.BlockSpec((1, 1, HID), lambda t, r: (t, 0, 0))
    return pl.pallas_call(
        _k2_body,
        grid=(2, N // R),
        in_specs=[pl.BlockSpec((1, R, HID), lambda t, r: (t, r, 0)),
                  wspec, wspec, wspec, bspec, bspec, bspec],
        out_specs=[pl.BlockSpec((1, R, HID), lambda t, r: (t, r, 0)),
                   pl.BlockSpec((1, 2 * R, HID), lambda t, r: (t, r, 0))],
        out_shape=[jax.ShapeDtypeStruct((2, N, HID), jnp.float32),
                   jax.ShapeDtypeStruct((2, 2 * N, HID), jnp.float32)],
    )(x, wq, wk, wv, bq, bk, bv)


def _k3_body(acc_ref, x_ref, w_ref, b_ref, al_ref, g_ref, bb_ref, o_ref):
    a0 = acc_ref[0, 0]
    a1 = acc_ref[0, 1]
    num = jnp.concatenate([a0[:, 0:128], a1[:, 0:128]], axis=1)
    den8 = jnp.concatenate([a0[:, 128:132], a1[:, 128:132]], axis=1)
    r8 = lax.broadcasted_iota(jnp.int32, (8, HID), 0)
    c8 = lax.broadcasted_iota(jnp.int32, (8, HID), 1) // DH
    expand = (r8 == c8).astype(jnp.float32)
    den = jnp.dot(den8, expand, preferred_element_type=jnp.float32)
    o = num / (den + 1e-16)
    o = 0.5 * o * (1.0 + lax.erf(o * 0.7071067811865475))
    o = jnp.dot(o, w_ref[0], preferred_element_type=jnp.float32) + b_ref[0]
    al = al_ref[0]
    o = al * o + (1.0 - al) * x_ref[0]
    o_ref[0] = g_ref[0] * o + bb_ref[0]


def _k3(acc, x, w, b, al, bng, bnb):
    return pl.pallas_call(
        _k3_body,
        grid=(2, N // R),
        in_specs=[
            pl.BlockSpec((1, 2, R, MW), lambda t, r: (t, 0, r, 0)),
            pl.BlockSpec((1, R, HID), lambda t, r: (t, r, 0)),
            pl.BlockSpec((1, HID, HID), lambda t, r: (t, 0, 0)),
            pl.BlockSpec((1, 1, HID), lambda t, r: (t, 0, 0)),
            pl.BlockSpec((1, 1, 1), lambda t, r: (t, 0, 0)),
            pl.BlockSpec((1, 1, HID), lambda t, r: (0, 0, 0)),
            pl.BlockSpec((1, 1, HID), lambda t, r: (0, 0, 0)),
        ],
        out_specs=pl.BlockSpec((1, R, HID), lambda t, r: (t, r, 0)),
        out_shape=jax.ShapeDtypeStruct((2, N, HID), jnp.float32),
    )(acc, x, w, b, al, bng, bnb)


def _edge_body(q_hbm, kv_hbm, ei_hbm, out_hbm,
               acc_sh, eib, ixd, ixkv, dsc, qr, kvr, msg,
               semI, semQ, semKV):
    c = lax.axis_index("c")
    s = lax.axis_index("s")

    lane = lax.iota(jnp.int32, 16)
    oh = [jnp.where(lane == h, 1.0, 0.0).astype(jnp.float32) for h in range(4)]

    z16 = jnp.zeros((16,), jnp.float32)

    def zrow(r, carry):
        for k2 in range(MW // 16):
            msg[r, pl.ds(k2 * 16, 16)] = z16
        return carry

    lax.fori_loop(0, C, zrow, 0)
    for j in range(RPT // C):
        pltpu.sync_copy(msg, acc_sh.at[pl.ds(s * RPT + j * C, C)])
    plsc.subcore_barrier()

    base = s * EPT

    def load_idx(ch, b):
        return pltpu.async_copy(
            ei_hbm.at[pl.ds(0, 2), pl.ds(base + ch * C, 48)], eib.at[b], semI)

    def comp_idx(b):
        for lo in (0, 16, 24):
            sv = eib[b, 0, pl.ds(lo, 16)]
            dv = eib[b, 1, pl.ds(lo, 16)]
            ixkv[b, pl.ds(lo, 16)] = sv + sv + c
            ixd[b, pl.ds(lo, 16)] = dv + dv + c
            dsc[b, pl.ds(lo, 16)] = dv

    def start_gathers(b):
        pltpu.async_copy(q_hbm.at[ixd.at[b]], qr.at[b], semQ)
        pltpu.async_copy(kv_hbm.at[ixkv.at[b]], kvr.at[b], semKV)

    load_idx(0, 0).wait()
    comp_idx(0)
    start_gathers(0)
    load_idx(1, 1)

    def chunk(ch, carry):
        b = lax.rem(ch, 2)
        bn = lax.rem(ch + 1, 2)

        @pl.when(ch + 1 < NCH)
        def _():
            pltpu.make_async_copy(
                ei_hbm.at[pl.ds(0, 2), pl.ds(base, 48)], eib.at[bn],
                semI).wait()
            comp_idx(bn)
            start_gathers(bn)

        @pl.when(ch + 2 < NCH)
        def _():
            load_idx(ch + 2, b)

        pltpu.make_async_copy(q_hbm.at[ixd.at[b]], qr.at[b], semQ).wait()
        pltpu.make_async_copy(kv_hbm.at[ixkv.at[b]], kvr.at[b], semKV).wait()

        @plsc.parallel_loop(0, C, unroll=4)
        def edge(e):
            dsum = None
            for h in range(4):
                qa = qr[b, e, pl.ds(h * 32, 16)]
                qb = qr[b, e, pl.ds(h * 32 + 16, 16)]
                ka = kvr[b, e, pl.ds(h * 32, 16)]
                kb = kvr[b, e, pl.ds(h * 32 + 16, 16)]
                sc = jnp.sum(qa * ka + qb * kb)
                w = jnp.exp(jnp.full((16,), sc, jnp.float32))
                msg[e, pl.ds(h * 32, 16)] = w * kvr[b, e, pl.ds(128 + h * 32, 16)]
                msg[e, pl.ds(h * 32 + 16, 16)] = w * kvr[b, e, pl.ds(128 + h * 32 + 16, 16)]
                part = w * oh[h]
                dsum = part if dsum is None else dsum + part
            msg[e, pl.ds(128, 16)] = dsum
        pltpu.sync_copy(msg, acc_sh.at[dsc.at[b]], add=True)
        return carry

    lax.fori_loop(0, NCH, chunk, 0)
    plsc.subcore_barrier()
    pltpu.sync_copy(acc_sh.at[pl.ds(s * RPT, RPT)],
                    out_hbm.at[c, pl.ds(s * RPT, RPT)])


_edge_call = pl.kernel(
    _edge_body,
    out_type=jax.ShapeDtypeStruct((2, NP, MW), jnp.float32),
    mesh=plsc.VectorSubcoreMesh(core_axis_name="c", subcore_axis_name="s"),
    compiler_params=pltpu.CompilerParams(use_tc_tiling_on_sc=False,
                                         needs_layout_passes=False),
    scratch_types=[
        pltpu.VMEM_SHARED((NP, MW), jnp.float32),
        pltpu.VMEM((2, 2, 48), jnp.int32),
        pltpu.VMEM((2, C), jnp.int32),
        pltpu.VMEM((2, C), jnp.int32),
        pltpu.VMEM((2, C), jnp.int32),
        pltpu.VMEM((2, C, 128), jnp.float32),
        pltpu.VMEM((2, C, HID), jnp.float32),
        pltpu.VMEM((C, MW), jnp.float32),
        pltpu.SemaphoreType.DMA,
        pltpu.SemaphoreType.DMA,
        pltpu.SemaphoreType.DMA,
    ],
)


def kernel(x_user, x_item, linW, linB, kW, kB, qW, qB, vW, vB, aW, aB,
           skip, relA, relM, relP, bnG, bnB, ei_ui, ei_iu):
    x0 = jnp.stack([x_user, x_item])
    xs = _k1(x0, linW, linB.reshape(2, 1, HID))

    pad = jnp.zeros((2, 64), jnp.int32)
    eis = [jnp.concatenate([ei_ui, pad], axis=1),
           jnp.concatenate([ei_iu, pad], axis=1)]

    for l in range(NLAYER):
        scale = relP[l] / jnp.sqrt(jnp.float32(DH))
        kWf = jnp.einsum('tihd,thde,th->tihe',
                         kW[l].reshape(2, HID, H, DH), relA[l], scale
                         ).reshape(2, HID, HID)
        kBf = jnp.einsum('thd,thde,th->the',
                         kB[l].reshape(2, H, DH), relA[l], scale
                         ).reshape(2, 1, HID)
        vWf = jnp.einsum('tihd,thde->tihe',
                         vW[l].reshape(2, HID, H, DH), relM[l]
                         ).reshape(2, HID, HID)
        vBf = jnp.einsum('thd,thde->the',
                         vB[l].reshape(2, H, DH), relM[l]
                         ).reshape(2, 1, HID)
        Q, KV = _k2(xs, qW[l], kWf, vWf,
                    qB[l].reshape(2, 1, HID), kBf, vBf)
        q2 = Q.reshape(2, 2 * N, 128)
        acc_item = _edge_call(q2[1], KV[0], eis[0])
        acc_user = _edge_call(q2[0], KV[1], eis[1])
        acc = jnp.stack([acc_user, acc_item])
        alpha = jax.nn.sigmoid(skip[l]).reshape(2, 1, 1)
        bng = (bnG[l] / jnp.sqrt(1.0 + 1e-5)).reshape(1, 1, HID)
        bnb = bnB[l].reshape(1, 1, HID)
        xs = _k3(acc, xs, aW[l], aB[l].reshape(2, 1, HID), alpha, bng, bnb)
    return xs


# speedup vs baseline: 55.2587x; 1.0475x over previous
import jax
import jax.numpy as jnp
from jax import lax
from jax.experimental import pallas as pl
from jax.experimental.pallas import tpu as pltpu
from jax.experimental.pallas import tpu_sc as plsc

N = 10000
E = 160000
D_IN = 128
H = 8
HID = 256
DH = 32
NLAYER = 2
MW = 144
C = 40
HALVES = ((0, 24), (24, 16))
NTILE = 16
EPT = E // NTILE
NCH = EPT // C
NP = 10240
RPT = NP // NTILE
R = 2000


def _k1_body(x_ref, w_ref, b_ref, o_ref):
    o_ref[0] = jax.nn.relu(
        jnp.dot(x_ref[0], w_ref[0], preferred_element_type=jnp.float32)
        + b_ref[0])


def _k1(x, w, b):
    return pl.pallas_call(
        _k1_body,
        grid=(2, N // R),
        in_specs=[
            pl.BlockSpec((1, R, D_IN), lambda t, r: (t, r, 0)),
            pl.BlockSpec((1, D_IN, HID), lambda t, r: (t, 0, 0)),
            pl.BlockSpec((1, 1, HID), lambda t, r: (t, 0, 0)),
        ],
        out_specs=pl.BlockSpec((1, R, HID), lambda t, r: (t, r, 0)),
        out_shape=jax.ShapeDtypeStruct((2, N, HID), jnp.float32),
    )(x, w, b)


def _k2_body(x_ref, wq_ref, wk_ref, wv_ref, bq_ref, bk_ref, bv_ref,
             q_ref, kv_ref):
    x = x_ref[0]
    q_ref[0] = jnp.dot(x, wq_ref[0], preferred_element_type=jnp.float32) + bq_ref[0]
    k = jnp.dot(x, wk_ref[0], preferred_element_type=jnp.float32) + bk_ref[0]
    v = jnp.dot(x, wv_ref[0], preferred_element_type=jnp.float32) + bv_ref[0]
    kv = jnp.concatenate([k.reshape(R, 2, 128), v.reshape(R, 2, 128)], axis=2)
    kv_ref[0] = kv.reshape(2 * R, HID)


def _k2(x, wq, wk, wv, bq, bk, bv):
    wspec = pl.BlockSpec((1, HID, HID), lambda t, r: (t, 0, 0))
    bspec = pl.BlockSpec((1, 1, HID), lambda t, r: (t, 0, 0))
    return pl.pallas_call(
        _k2_body,
        grid=(2, N // R),
        in_specs=[pl.BlockSpec((1, R, HID), lambda t, r: (t, r, 0)),
                  wspec, wspec, wspec, bspec, bspec, bspec],
        out_specs=[pl.BlockSpec((1, R, HID), lambda t, r: (t, r, 0)),
                   pl.BlockSpec((1, 2 * R, HID), lambda t, r: (t, r, 0))],
        out_shape=[jax.ShapeDtypeStruct((2, N, HID), jnp.float32),
                   jax.ShapeDtypeStruct((2, 2 * N, HID), jnp.float32)],
    )(x, wq, wk, wv, bq, bk, bv)


def _k3_body(acc_ref, x_ref, w_ref, b_ref, al_ref, g_ref, bb_ref, o_ref):
    a0 = acc_ref[0, 0]
    a1 = acc_ref[0, 1]
    num = jnp.concatenate([a0[:, 0:128], a1[:, 0:128]], axis=1)
    den8 = jnp.concatenate([a0[:, 128:132], a1[:, 128:132]], axis=1)
    r8 = lax.broadcasted_iota(jnp.int32, (8, HID), 0)
    c8 = lax.broadcasted_iota(jnp.int32, (8, HID), 1) // DH
    expand = (r8 == c8).astype(jnp.float32)
    den = jnp.dot(den8, expand, preferred_element_type=jnp.float32)
    o = num / (den + 1e-16)
    o = 0.5 * o * (1.0 + lax.erf(o * 0.7071067811865475))
    o = jnp.dot(o, w_ref[0], preferred_element_type=jnp.float32) + b_ref[0]
    al = al_ref[0]
    o = al * o + (1.0 - al) * x_ref[0]
    o_ref[0] = g_ref[0] * o + bb_ref[0]


def _k3(acc, x, w, b, al, bng, bnb):
    return pl.pallas_call(
        _k3_body,
        grid=(2, N // R),
        in_specs=[
            pl.BlockSpec((1, 2, R, MW), lambda t, r: (t, 0, r, 0)),
            pl.BlockSpec((1, R, HID), lambda t, r: (t, r, 0)),
            pl.BlockSpec((1, HID, HID), lambda t, r: (t, 0, 0)),
            pl.BlockSpec((1, 1, HID), lambda t, r: (t, 0, 0)),
            pl.BlockSpec((1, 1, 1), lambda t, r: (t, 0, 0)),
            pl.BlockSpec((1, 1, HID), lambda t, r: (0, 0, 0)),
            pl.BlockSpec((1, 1, HID), lambda t, r: (0, 0, 0)),
        ],
        out_specs=pl.BlockSpec((1, R, HID), lambda t, r: (t, r, 0)),
        out_shape=jax.ShapeDtypeStruct((2, N, HID), jnp.float32),
    )(acc, x, w, b, al, bng, bnb)


def _edge_body(q_hbm, kv_hbm, ei_hbm, out_hbm,
               acc_sh, eib, ixd, ixkv, dsc, qr, kvr, msg,
               semI, semQ, semKV, semS):
    c = lax.axis_index("c")
    s = lax.axis_index("s")

    lane = lax.iota(jnp.int32, 16)
    oh = [jnp.where(lane == h, 1.0, 0.0).astype(jnp.float32) for h in range(4)]

    z16 = jnp.zeros((16,), jnp.float32)

    def zrow(r, carry):
        for k2 in range(MW // 16):
            msg[r, pl.ds(k2 * 16, 16)] = z16
        return carry

    lax.fori_loop(0, C, zrow, 0)
    for j in range(RPT // C):
        pltpu.sync_copy(msg, acc_sh.at[pl.ds(s * RPT + j * C, C)])
    plsc.subcore_barrier()

    base = s * EPT

    def load_idx(ch, b):
        return pltpu.async_copy(
            ei_hbm.at[pl.ds(0, 2), pl.ds(base + ch * C, 48)], eib.at[b], semI)

    def comp_idx(b, d3):
        for lo in (0, 16, 24):
            sv = eib[b, 0, pl.ds(lo, 16)]
            dv = eib[b, 1, pl.ds(lo, 16)]
            ixkv[b, pl.ds(lo, 16)] = sv + sv + c
            ixd[b, pl.ds(lo, 16)] = dv + dv + c
            dsc[d3, pl.ds(lo, 16)] = dv

    def start_gathers(b):
        pltpu.async_copy(q_hbm.at[ixd.at[b]], qr.at[b], semQ)
        pltpu.async_copy(kv_hbm.at[ixkv.at[b]], kvr.at[b], semKV)

    load_idx(0, 0).wait()
    comp_idx(0, 0)
    start_gathers(0)
    load_idx(1, 1)

    def chunk(ch, carry):
        b = lax.rem(ch, 2)
        bn = lax.rem(ch + 1, 2)
        b3 = lax.rem(ch, 3)
        d3n = lax.rem(ch + 1, 3)

        @pl.when(ch + 1 < NCH)
        def _():
            pltpu.make_async_copy(
                ei_hbm.at[pl.ds(0, 2), pl.ds(base, 48)], eib.at[bn],
                semI).wait()
            comp_idx(bn, d3n)
            start_gathers(bn)

        @pl.when(ch + 2 < NCH)
        def _():
            load_idx(ch + 2, b)

        pltpu.make_async_copy(q_hbm.at[ixd.at[b]], qr.at[b], semQ).wait()
        pltpu.make_async_copy(kv_hbm.at[ixkv.at[b]], kvr.at[b], semKV).wait()

        for j, (off, hc) in enumerate(HALVES):
            @pl.when(ch >= 1)
            def _():
                pltpu.make_async_copy(
                    msg.at[pl.ds(off, hc)],
                    acc_sh.at[dsc.at[b3, pl.ds(off, hc)]], semS.at[j]).wait()

            @plsc.parallel_loop(0, hc, unroll=4)
            def edge(e0):
                e = off + e0
                dsum = None
                for h in range(4):
                    qa = qr[b, e, pl.ds(h * 32, 16)]
                    qb = qr[b, e, pl.ds(h * 32 + 16, 16)]
                    ka = kvr[b, e, pl.ds(h * 32, 16)]
                    kb = kvr[b, e, pl.ds(h * 32 + 16, 16)]
                    sc = jnp.sum(qa * ka + qb * kb)
                    w = jnp.exp(jnp.full((16,), sc, jnp.float32))
                    msg[e, pl.ds(h * 32, 16)] = w * kvr[b, e, pl.ds(128 + h * 32, 16)]
                    msg[e, pl.ds(h * 32 + 16, 16)] = w * kvr[b, e, pl.ds(128 + h * 32 + 16, 16)]
                    part = w * oh[h]
                    dsum = part if dsum is None else dsum + part
                msg[e, pl.ds(128, 16)] = dsum
            pltpu.async_copy(msg.at[pl.ds(off, hc)],
                             acc_sh.at[dsc.at[b3, pl.ds(off, hc)]],
                             semS.at[j], add=True)
        return carry

    lax.fori_loop(0, NCH, chunk, 0)
    for j, (off, hc) in enumerate(HALVES):
        pltpu.make_async_copy(msg.at[pl.ds(off, hc)],
                              acc_sh.at[dsc.at[0, pl.ds(off, hc)]],
                              semS.at[j]).wait()
    plsc.subcore_barrier()
    pltpu.sync_copy(acc_sh.at[pl.ds(s * RPT, RPT)],
                    out_hbm.at[c, pl.ds(s * RPT, RPT)])


_edge_call = pl.kernel(
    _edge_body,
    out_type=jax.ShapeDtypeStruct((2, NP, MW), jnp.float32),
    mesh=plsc.VectorSubcoreMesh(core_axis_name="c", subcore_axis_name="s"),
    compiler_params=pltpu.CompilerParams(use_tc_tiling_on_sc=False,
                                         needs_layout_passes=False),
    scratch_types=[
        pltpu.VMEM_SHARED((NP, MW), jnp.float32),
        pltpu.VMEM((2, 2, 48), jnp.int32),
        pltpu.VMEM((2, C), jnp.int32),
        pltpu.VMEM((2, C), jnp.int32),
        pltpu.VMEM((3, C), jnp.int32),
        pltpu.VMEM((2, C, 128), jnp.float32),
        pltpu.VMEM((2, C, HID), jnp.float32),
        pltpu.VMEM((C, MW), jnp.float32),
        pltpu.SemaphoreType.DMA,
        pltpu.SemaphoreType.DMA,
        pltpu.SemaphoreType.DMA,
        pltpu.SemaphoreType.DMA((2,)),
    ],
)


def kernel(x_user, x_item, linW, linB, kW, kB, qW, qB, vW, vB, aW, aB,
           skip, relA, relM, relP, bnG, bnB, ei_ui, ei_iu):
    x0 = jnp.stack([x_user, x_item])
    xs = _k1(x0, linW, linB.reshape(2, 1, HID))

    pad = jnp.zeros((2, 64), jnp.int32)
    eis = [jnp.concatenate([ei_ui, pad], axis=1),
           jnp.concatenate([ei_iu, pad], axis=1)]

    for l in range(NLAYER):
        scale = relP[l] / jnp.sqrt(jnp.float32(DH))
        kWf = jnp.einsum('tihd,thde,th->tihe',
                         kW[l].reshape(2, HID, H, DH), relA[l], scale
                         ).reshape(2, HID, HID)
        kBf = jnp.einsum('thd,thde,th->the',
                         kB[l].reshape(2, H, DH), relA[l], scale
                         ).reshape(2, 1, HID)
        vWf = jnp.einsum('tihd,thde->tihe',
                         vW[l].reshape(2, HID, H, DH), relM[l]
                         ).reshape(2, HID, HID)
        vBf = jnp.einsum('thd,thde->the',
                         vB[l].reshape(2, H, DH), relM[l]
                         ).reshape(2, 1, HID)
        Q, KV = _k2(xs, qW[l], kWf, vWf,
                    qB[l].reshape(2, 1, HID), kBf, vBf)
        q2 = Q.reshape(2, 2 * N, 128)
        acc_item = _edge_call(q2[1], KV[0], eis[0])
        acc_user = _edge_call(q2[0], KV[1], eis[1])
        acc = jnp.stack([acc_user, acc_item])
        alpha = jax.nn.sigmoid(skip[l]).reshape(2, 1, 1)
        bng = (bnG[l] / jnp.sqrt(1.0 + 1e-5)).reshape(1, 1, HID)
        bnb = bnB[l].reshape(1, 1, HID)
        xs = _k3(acc, xs, aW[l], aB[l].reshape(2, 1, HID), alpha, bng, bnb)
    return xs
